# baseline (speedup 1.0000x reference)
import functools

import jax
import jax.numpy as jnp
from jax import lax
from jax.experimental import pallas as pl
from jax.experimental.pallas import tpu as pltpu
from jax.experimental.pallas import tpu_sc as plsc

M = 4096
A = 200
D = 32
N = M * A
V = 1000000
NC = 2
NS = 16
NW = NC * NS

SUP = 1024
NSUP = V // SUP
TAIL0 = NSUP * SUP
TAIL_N = 512
TAIL2 = V - TAIL0 - TAIL_N

CH = 1024
NCH = N // CH
CPW = NCH // NW


def _iota16():
    return lax.iota(jnp.int32, 16)


def _detile_body(tabt_hbm, tail_hbm, lin_hbm, stage_v, lin_v, sem):
    wid = lax.axis_index("s") * NC + lax.axis_index("c")
    iota = _iota16()
    iota32 = iota * 32

    def do_super(s):
        l0 = s * SUP
        cps = []
        for q in range(4):
            for lt in range(8):
                cps.append(pltpu.async_copy(
                    tabt_hbm.at[pl.ds(q * 8, 8), pl.ds(l0 + lt * 128, 128)],
                    stage_v.at[lt, pl.ds(q * 8, 8), :], sem))
        for cp in cps:
            cp.wait()

        @plsc.parallel_loop(0, SUP // 16, 1, unroll=2)
        def _(li):
            lt = li >> 3
            lo = (li & 7) * 16
            base = iota32 + li * 512
            for c in range(32):
                v = stage_v[lt, c, pl.ds(lo, 16)]
                plsc.store_scatter(lin_v, [base + c], v)

        pltpu.sync_copy(lin_v, lin_hbm.at[pl.ds(l0 * 32, SUP * 32)])

    def round_g(g, _):
        s = g * NW + wid

        @pl.when(s < NSUP)
        def _():
            do_super(s)
        return ()

    lax.fori_loop(0, (NSUP + NW - 1) // NW, round_g, ())

    @pl.when(wid == NW - 1)
    def _():
        cps = []
        for q in range(4):
            for lt in range(4):
                cps.append(pltpu.async_copy(
                    tabt_hbm.at[pl.ds(q * 8, 8), pl.ds(TAIL0 + lt * 128, 128)],
                    stage_v.at[lt, pl.ds(q * 8, 8), :], sem))
        for cp in cps:
            cp.wait()

        @plsc.parallel_loop(0, TAIL_N // 16, 1, unroll=2)
        def _(li):
            lt = li >> 3
            lo = (li & 7) * 16
            base = iota32 + li * 512
            for c in range(32):
                v = stage_v[lt, c, pl.ds(lo, 16)]
                plsc.store_scatter(lin_v, [base + c], v)

        pltpu.sync_copy(lin_v.at[pl.ds(0, TAIL_N * 32)],
                        lin_hbm.at[pl.ds(TAIL0 * 32, TAIL_N * 32)])

    @pl.when(wid == NW - 2)
    def _():
        pltpu.sync_copy(tail_hbm, lin_v.at[pl.ds(0, TAIL2 * 32)])
        pltpu.sync_copy(lin_v.at[pl.ds(0, TAIL2 * 32)],
                        lin_hbm.at[pl.ds((TAIL0 + TAIL_N) * 32, TAIL2 * 32)])


def _gather_body(x_hbm, pos_hbm, tab_hbm, w_hbm, b_hbm, out_hbm,
                 idx_v, pos_v, rows_v, out_v, wb_v, sem):
    wid = lax.axis_index("s") * NC + lax.axis_index("c")
    iota = _iota16()

    pltpu.sync_copy(w_hbm, wb_v.at[pl.ds(0, 96)])
    pltpu.sync_copy(b_hbm, wb_v.at[pl.ds(96, 32)])
    wvec = [wb_v[pl.ds(k * 16, 16)] for k in range(8)]
    wsc = [[wvec[2 * j + (c >> 4)][c & 15] for c in range(32)]
           for j in range(4)]

    def do_chunk(g, _):
        cid = g * NW + wid
        a = cid // 4
        i0 = (cid % 4) * CH
        tok0 = a * M + i0
        pltpu.sync_copy(x_hbm.at[pl.ds(tok0, CH)], idx_v)
        for j in range(3):
            pltpu.sync_copy(pos_hbm.at[pl.ds(j * N + tok0, CH)],
                            pos_v.at[pl.ds(j * CH, CH)])
        cps = []
        for s in range(CH // 128):
            cps.append(pltpu.async_copy(
                tab_hbm.at[idx_v.at[pl.ds(s * 128, 128)]],
                rows_v.at[pl.ds(s * 128, 128), :], sem))
        for cp in cps:
            cp.wait()

        for c in range(32):
            w0v = jnp.broadcast_to(wsc[0][c], (16,))
            w1v = jnp.broadcast_to(wsc[1][c], (16,))
            w2v = jnp.broadcast_to(wsc[2][c], (16,))
            bv = jnp.broadcast_to(wsc[3][c], (16,))
            cvec = jnp.full((16,), c, jnp.int32)

            def do_i(i, _, w0v=w0v, w1v=w1v, w2v=w2v, bv=bv, cvec=cvec, c=c):
                tok = iota + i * 16
                p0 = pos_v[pl.ds(i * 16, 16)]
                p1 = pos_v[pl.ds(CH + i * 16, 16)]
                p2 = pos_v[pl.ds(2 * CH + i * 16, 16)]
                gth = plsc.load_gather(rows_v, [tok, cvec])
                out_v[c, pl.ds(i * 16, 16)] = (
                    gth + ((p0 * w0v + p1 * w1v) + (p2 * w2v + bv)))
                return ()

            lax.fori_loop(0, CH // 16, do_i, (), unroll=4)
        pltpu.sync_copy(out_v, out_hbm.at[a, :, pl.ds(i0, CH)])
        return ()

    lax.fori_loop(0, CPW, do_chunk, ())


@jax.jit
def kernel(x, positions, token_table, W, b):
    tabt = token_table.T
    xt = x.T.reshape(N)
    post = positions.transpose(2, 1, 0).reshape(3 * N)
    wflat = W.reshape(3 * D)

    detile = pl.kernel(
        _detile_body,
        out_type=jax.ShapeDtypeStruct((V * D,), jnp.float32),
        mesh=plsc.VectorSubcoreMesh(core_axis_name="c", subcore_axis_name="s"),
        scratch_types=[
            pltpu.VMEM((8, 32, 128), jnp.float32),
            pltpu.VMEM((SUP * 32,), jnp.float32),
            pltpu.SemaphoreType.DMA,
        ],
        compiler_params=pltpu.CompilerParams(needs_layout_passes=False),
    )
    tail = token_table[TAIL0 + TAIL_N:].reshape(TAIL2 * D)
    lin = detile(tabt, tail)
    tab2 = lin.reshape(V, D)

    gather = pl.kernel(
        _gather_body,
        out_type=jax.ShapeDtypeStruct((A, D, M), jnp.float32),
        mesh=plsc.VectorSubcoreMesh(core_axis_name="c", subcore_axis_name="s"),
        scratch_types=[
            pltpu.VMEM((CH,), jnp.int32),
            pltpu.VMEM((3 * CH,), jnp.float32),
            pltpu.VMEM((CH, D), jnp.float32),
            pltpu.VMEM((D, CH), jnp.float32),
            pltpu.VMEM((128,), jnp.float32),
            pltpu.SemaphoreType.DMA,
        ],
        compiler_params=pltpu.CompilerParams(use_tc_tiling_on_sc=False,
                                             needs_layout_passes=False),
    )
    out = gather(xt, post, tab2, wflat, b)
    return out.transpose(2, 0, 1)


# speedup vs baseline: 1.1744x; 1.1744x over previous
import functools

import jax
import jax.numpy as jnp
from jax import lax
from jax.experimental import pallas as pl
from jax.experimental.pallas import tpu as pltpu
from jax.experimental.pallas import tpu_sc as plsc

M = 4096
A = 200
D = 32
N = M * A
V = 1000000
NC = 2
NS = 16
NW = NC * NS

SUP = 1024
NSUP = V // SUP
TAIL0 = NSUP * SUP
TAIL_N = 512
TAIL2 = V - TAIL0 - TAIL_N

CH = 1024
NCH = N // CH
CPW = NCH // NW


def _iota16():
    return lax.iota(jnp.int32, 16)


def _detile_body(tabt_hbm, tail_hbm, lin_hbm, stage_v, lin_v, sem):
    wid = lax.axis_index("s") * NC + lax.axis_index("c")
    iota = _iota16()
    iota32 = iota * 32

    def do_super(s):
        l0 = s * SUP
        cps = []
        for q in range(4):
            for lt in range(8):
                cps.append(pltpu.async_copy(
                    tabt_hbm.at[pl.ds(q * 8, 8), pl.ds(l0 + lt * 128, 128)],
                    stage_v.at[lt, pl.ds(q * 8, 8), :], sem))
        for cp in cps:
            cp.wait()

        @plsc.parallel_loop(0, SUP // 16, 1, unroll=2)
        def _(li):
            lt = li >> 3
            lo = (li & 7) * 16
            base = iota32 + li * 512
            for c in range(32):
                v = stage_v[lt, c, pl.ds(lo, 16)]
                plsc.store_scatter(lin_v, [base + c], v)

        pltpu.sync_copy(lin_v, lin_hbm.at[pl.ds(l0 * 32, SUP * 32)])

    def round_g(g, _):
        s = g * NW + wid

        @pl.when(s < NSUP)
        def _():
            do_super(s)
        return ()

    lax.fori_loop(0, (NSUP + NW - 1) // NW, round_g, ())

    @pl.when(wid == NW - 1)
    def _():
        cps = []
        for q in range(4):
            for lt in range(4):
                cps.append(pltpu.async_copy(
                    tabt_hbm.at[pl.ds(q * 8, 8), pl.ds(TAIL0 + lt * 128, 128)],
                    stage_v.at[lt, pl.ds(q * 8, 8), :], sem))
        for cp in cps:
            cp.wait()

        @plsc.parallel_loop(0, TAIL_N // 16, 1, unroll=2)
        def _(li):
            lt = li >> 3
            lo = (li & 7) * 16
            base = iota32 + li * 512
            for c in range(32):
                v = stage_v[lt, c, pl.ds(lo, 16)]
                plsc.store_scatter(lin_v, [base + c], v)

        pltpu.sync_copy(lin_v.at[pl.ds(0, TAIL_N * 32)],
                        lin_hbm.at[pl.ds(TAIL0 * 32, TAIL_N * 32)])

    @pl.when(wid == NW - 2)
    def _():
        pltpu.sync_copy(tail_hbm, lin_v.at[pl.ds(0, TAIL2 * 32)])
        pltpu.sync_copy(lin_v.at[pl.ds(0, TAIL2 * 32)],
                        lin_hbm.at[pl.ds((TAIL0 + TAIL_N) * 32, TAIL2 * 32)])


def _gather_body(x_hbm, pos_hbm, tab_hbm, w_hbm, b_hbm, out_hbm,
                 idx_v, pos_v, rows_v, out_v, wb_v, sem):
    wid = lax.axis_index("s") * NC + lax.axis_index("c")
    iota = _iota16()

    pltpu.sync_copy(w_hbm, wb_v.at[pl.ds(0, 96)])
    pltpu.sync_copy(b_hbm, wb_v.at[pl.ds(96, 32)])
    wvec = [wb_v[pl.ds(k * 16, 16)] for k in range(8)]
    wsc = [[wvec[2 * j + (c >> 4)][c & 15] for c in range(32)]
           for j in range(4)]

    def do_chunk(g, _):
        cid = g * NW + wid
        a = cid // 4
        i0 = (cid % 4) * CH
        tok0 = a * M + i0
        pltpu.sync_copy(x_hbm.at[pl.ds(tok0, CH)], idx_v)
        for j in range(3):
            pltpu.sync_copy(pos_hbm.at[pl.ds(j * N + tok0, CH)],
                            pos_v.at[pl.ds(j * CH, CH)])
        cps = []
        for s in range(CH // 128):
            cps.append(pltpu.async_copy(
                tab_hbm.at[idx_v.at[pl.ds(s * 128, 128)]],
                rows_v.at[pl.ds(s * 128, 128), :], sem))
        for cp in cps:
            cp.wait()

        def do_i(i, _):
            tok = iota + i * 16
            p0 = pos_v[pl.ds(i * 16, 16)]
            p1 = pos_v[pl.ds(CH + i * 16, 16)]
            p2 = pos_v[pl.ds(2 * CH + i * 16, 16)]
            for cg in range(4):
                gth = [plsc.load_gather(
                           rows_v, [tok, jnp.full((16,), cg * 8 + k, jnp.int32)])
                       for k in range(8)]
                for k in range(8):
                    c = cg * 8 + k
                    t0 = p0 * wsc[0][c] + p1 * wsc[1][c]
                    t1 = p2 * wsc[2][c] + wsc[3][c]
                    out_v[c, pl.ds(i * 16, 16)] = gth[k] + (t0 + t1)
            return ()

        lax.fori_loop(0, CH // 16, do_i, ())
        pltpu.sync_copy(out_v, out_hbm.at[a, :, pl.ds(i0, CH)])
        return ()

    lax.fori_loop(0, CPW, do_chunk, ())


@jax.jit
def kernel(x, positions, token_table, W, b):
    tabt = token_table.T
    xt = x.T.reshape(N)
    post = positions.transpose(2, 1, 0).reshape(3 * N)
    wflat = W.reshape(3 * D)

    detile = pl.kernel(
        _detile_body,
        out_type=jax.ShapeDtypeStruct((V * D,), jnp.float32),
        mesh=plsc.VectorSubcoreMesh(core_axis_name="c", subcore_axis_name="s"),
        scratch_types=[
            pltpu.VMEM((8, 32, 128), jnp.float32),
            pltpu.VMEM((SUP * 32,), jnp.float32),
            pltpu.SemaphoreType.DMA,
        ],
        compiler_params=pltpu.CompilerParams(needs_layout_passes=False),
    )
    tail = token_table[TAIL0 + TAIL_N:].reshape(TAIL2 * D)
    lin = detile(tabt, tail)
    tab2 = lin.reshape(V, D)

    gather = pl.kernel(
        _gather_body,
        out_type=jax.ShapeDtypeStruct((A, D, M), jnp.float32),
        mesh=plsc.VectorSubcoreMesh(core_axis_name="c", subcore_axis_name="s"),
        scratch_types=[
            pltpu.VMEM((CH,), jnp.int32),
            pltpu.VMEM((3 * CH,), jnp.float32),
            pltpu.VMEM((CH, D), jnp.float32),
            pltpu.VMEM((D, CH), jnp.float32),
            pltpu.VMEM((128,), jnp.float32),
            pltpu.SemaphoreType.DMA,
        ],
        compiler_params=pltpu.CompilerParams(use_tc_tiling_on_sc=False,
                                             needs_layout_passes=False),
    )
    out = gather(xt, post, tab2, wflat, b)
    return out.transpose(2, 0, 1)
